# baseline (speedup 1.0000x reference)
import functools

import jax
import jax.numpy as jnp
from jax import lax
from jax.experimental import pallas as pl
from jax.experimental.pallas import tpu as pltpu
from jax.experimental.pallas import tpu_sc as plsc

_N = 50000
_E = 800000
_T = 12
_F_IN = 2
_D_EMB = 32
_H = 2
_C = 16
_D_GAT = _H * _C
_D_HID = 16

_ROW = 34
_NSUB = 16
_NCORE = 2
_T_PER_CORE = _T // _NCORE
_N_PAD = 50176
_RPS = _N_PAD // (_NSUB * _NCORE)
_E_PAD = 819200
_K = 256
_KB = _K // 128
_NCH = _E_PAD // _NSUB // _K


def _tables_body(x_ref, wemb_ref, bemb_ref, wgat_ref, asrc_ref, adst_ref,
                 srctab_ref, adtab_ref):
    xb = x_ref[...]
    for t in range(_T):
        xt = xb[:, _F_IN * t:_F_IN * (t + 1)]
        emb = jnp.maximum(
            jnp.dot(xt, wemb_ref[...], preferred_element_type=jnp.float32)
            + bemb_ref[...], 0.0)
        xl = jnp.dot(emb, wgat_ref[...], preferred_element_type=jnp.float32)
        a_s = jnp.dot(xl, asrc_ref[...], preferred_element_type=jnp.float32)
        a_d = jnp.dot(xl, adst_ref[...], preferred_element_type=jnp.float32)
        srctab_ref[t, :, 0:_D_GAT] = xl
        srctab_ref[t, :, _D_GAT:_ROW] = a_s
        adtab_ref[t, :, :] = a_d


def _build_tables(x2d, W_emb, b_emb, W_gat, As, Ad):
    B1 = 512
    nb = pl.cdiv(_N, B1)
    return pl.pallas_call(
        _tables_body,
        grid=(nb,),
        in_specs=[
            pl.BlockSpec((B1, _T * _F_IN), lambda i: (i, 0)),
            pl.BlockSpec((_F_IN, _D_EMB), lambda i: (0, 0)),
            pl.BlockSpec((1, _D_EMB), lambda i: (0, 0)),
            pl.BlockSpec((_D_EMB, _D_GAT), lambda i: (0, 0)),
            pl.BlockSpec((_D_GAT, _H), lambda i: (0, 0)),
            pl.BlockSpec((_D_GAT, _H), lambda i: (0, 0)),
        ],
        out_specs=[
            pl.BlockSpec((_T, B1, _ROW), lambda i: (0, i, 0)),
            pl.BlockSpec((_T, B1, _H), lambda i: (0, i, 0)),
        ],
        out_shape=[
            jax.ShapeDtypeStruct((_T, _N, _ROW), jnp.float32),
            jax.ShapeDtypeStruct((_T, _N, _H), jnp.float32),
        ],
    )(x2d, W_emb, b_emb, W_gat, As, Ad)


_PROBE_NO_SCATTER = False
_PROBE_NO_COMPUTE = False
_PROBE_NO_GATHER = False
_KS = 256
_JROWS = _E_PAD // _NSUB // 128
_NSLOT = _JROWS // 2
_BLK = 8


def _sc_edge_pass(srctab_flat, adtab_flat, src2d, dst2d, zrows):
    mesh = plsc.VectorSubcoreMesh(core_axis_name="c", subcore_axis_name="s")

    @functools.partial(
        pl.kernel,
        out_type=jax.ShapeDtypeStruct((_T * _N_PAD, _ROW), jnp.float32),
        mesh=mesh,
        compiler_params=pltpu.CompilerParams(needs_layout_passes=False,
                                             use_tc_tiling_on_sc=False,
                                             disable_bounds_checks=True),
        scratch_types=[
            pltpu.VMEM((_BLK, 128), jnp.int32),
            pltpu.VMEM((_BLK, 128), jnp.int32),
            pltpu.VMEM((_KS,), jnp.int32),
            pltpu.VMEM((_KS,), jnp.int32),
            pltpu.VMEM((_KS,), jnp.int32),
            pltpu.VMEM((_KS, _ROW), jnp.float32),
            pltpu.VMEM((_KS, _H), jnp.float32),
            pltpu.VMEM_SHARED((_N_PAD, _ROW), jnp.float32),
            pltpu.SemaphoreType.DMA,
            pltpu.SemaphoreType.DMA,
        ],
    )
    def k(srctab_hbm, adtab_hbm, src2d_hbm, dst2d_hbm, z_hbm, out_hbm,
          srcblk, dstblk, sidx, didx, draw, sbuf, adrows, acc, semg, sems):
        cid = lax.axis_index("c")
        sid = lax.axis_index("s")
        lanes = lax.iota(jnp.int32, 16)

        def shift(r0, tshift):
            def body(i, _):
                f = i * 16 + lanes
                rvec = r0 + lax.shift_right_logical(f, 7)
                cvec = lax.bitwise_and(f, 127)
                s = plsc.load_gather(srcblk, [rvec, cvec])
                d = plsc.load_gather(dstblk, [rvec, cvec])
                sidx[pl.ds(i * 16, 16)] = s + tshift
                didx[pl.ds(i * 16, 16)] = d + tshift
                draw[pl.ds(i * 16, 16)] = d
                return 0
            lax.fori_loop(0, _KS // 16, body, 0)

        def issue_gather():
            pltpu.async_copy(srctab_hbm.at[sidx], sbuf, semg)
            pltpu.async_copy(adtab_hbm.at[didx], adrows, semg)

        def wait_gather():
            pltpu.make_async_copy(srctab_hbm.at[sidx], sbuf, semg).wait()
            pltpu.make_async_copy(adtab_hbm.at[didx], adrows, semg).wait()

        def issue_scatter():
            pltpu.async_copy(sbuf, acc.at[draw], sems, add=True)

        def wait_scatter():
            pltpu.make_async_copy(sbuf, acc.at[draw], sems).wait()

        def compute():
            sb = sbuf
            ab = adrows

            def body(i, _):
                e16 = i * 16 + lanes
                c32 = jnp.full((16,), _D_GAT, jnp.int32)
                c33 = jnp.full((16,), _D_GAT + 1, jnp.int32)
                as0 = plsc.load_gather(sb, [e16, c32])
                as1 = plsc.load_gather(sb, [e16, c33])
                ad0 = plsc.load_gather(ab, [e16, jnp.full((16,), 0, jnp.int32)])
                ad1 = plsc.load_gather(ab, [e16, jnp.full((16,), 1, jnp.int32)])
                al0 = as0 + ad0
                al1 = as1 + ad1
                al0 = jnp.where(al0 >= 0.0, al0, 0.2 * al0)
                al1 = jnp.where(al1 >= 0.0, al1, 0.2 * al1)
                ex0 = jnp.exp(al0)
                ex1 = jnp.exp(al1)
                for j in range(_D_GAT):
                    cj = jnp.full((16,), j, jnp.int32)
                    v = plsc.load_gather(sb, [e16, cj])
                    exj = ex0 if j < _C else ex1
                    plsc.store_scatter(sb, [e16, cj], v * exj)
                plsc.store_scatter(sb, [e16, c32], ex0)
                plsc.store_scatter(sb, [e16, c33], ex1)
                return 0

            lax.fori_loop(0, _KS // 16, body, 0)

        def fetch_block(b):
            irow = sid * _JROWS + b * _BLK
            pltpu.sync_copy(src2d_hbm.at[pl.ds(irow, _BLK)], srcblk)
            pltpu.sync_copy(dst2d_hbm.at[pl.ds(irow, _BLK)], dstblk)

        def per_t(tl, _):
            tg = cid * _T_PER_CORE + tl
            tshift = tg * _N
            pltpu.sync_copy(z_hbm, acc.at[pl.ds(sid * _RPS, _RPS)])
            plsc.subcore_barrier()

            def slot_body(ci, _):
                r0 = lax.rem(2 * ci, _BLK)

                if not _PROBE_NO_SCATTER:
                    @pl.when(ci > 0)
                    def _():
                        wait_scatter()

                @pl.when(r0 == 0)
                def _():
                    fetch_block((2 * ci) // _BLK)

                shift(r0, tshift)
                if not _PROBE_NO_GATHER:
                    issue_gather()
                    wait_gather()
                if not _PROBE_NO_COMPUTE:
                    compute()
                if _PROBE_NO_SCATTER:
                    pass
                else:
                    issue_scatter()
                return 0

            lax.fori_loop(0, _NSLOT, slot_body, 0)
            if not _PROBE_NO_SCATTER:
                wait_scatter()
            plsc.subcore_barrier()
            pltpu.sync_copy(
                acc.at[pl.ds(sid * _RPS, _RPS)],
                out_hbm.at[pl.ds(tg * _N_PAD + sid * _RPS, _RPS)])
            return 0

        lax.fori_loop(0, _T_PER_CORE, per_t, 0)

    return k(srctab_flat, adtab_flat, src2d, dst2d, zrows)


def _final_body(out_ref, srctab_ref, adtab_ref, bgat_ref, wihT_ref, whhT_ref,
                bih_ref, bhh_ref, wout_ref, bout_ref, pred_ref):
    B2 = pred_ref.shape[0]
    h = jnp.zeros((B2, _D_HID), jnp.float32)
    bgat = bgat_ref[...]
    for t in range(_T):
        row = out_ref[t]
        st = srctab_ref[t]
        num = row[:, 0:_D_GAT]
        den_e = row[:, _D_GAT:_ROW]
        xl = st[:, 0:_D_GAT]
        a_s = st[:, _D_GAT:_ROW]
        a_d = adtab_ref[t]
        alpha = a_s + a_d
        alpha = jnp.where(alpha >= 0.0, alpha, 0.2 * alpha)
        ex = jnp.exp(alpha)
        den = den_e + ex + 1e-16
        exb = jnp.concatenate(
            [jnp.broadcast_to(ex[:, 0:1], (B2, _C)),
             jnp.broadcast_to(ex[:, 1:2], (B2, _C))], axis=1)
        denb = jnp.concatenate(
            [jnp.broadcast_to(den[:, 0:1], (B2, _C)),
             jnp.broadcast_to(den[:, 1:2], (B2, _C))], axis=1)
        gat = (num + exb * xl) / denb + bgat
        gi = jnp.dot(gat, wihT_ref[...], preferred_element_type=jnp.float32) + bih_ref[...]
        gh = jnp.dot(h, whhT_ref[...], preferred_element_type=jnp.float32) + bhh_ref[...]
        r = jax.nn.sigmoid(gi[:, 0:_D_HID] + gh[:, 0:_D_HID])
        z = jax.nn.sigmoid(gi[:, _D_HID:2 * _D_HID] + gh[:, _D_HID:2 * _D_HID])
        ng = jnp.tanh(gi[:, 2 * _D_HID:3 * _D_HID] + r * gh[:, 2 * _D_HID:3 * _D_HID])
        h = (1.0 - z) * ng + z * h
    pred_ref[...] = jnp.dot(h, wout_ref[...], preferred_element_type=jnp.float32) + bout_ref[...]


def _final_stage(out3d, srctab, adtab, b_gat, wihT, whhT, b_ih, b_hh, W_out, b_out):
    B2 = 256
    nb = _N_PAD // B2
    return pl.pallas_call(
        _final_body,
        grid=(nb,),
        in_specs=[
            pl.BlockSpec((_T, B2, _ROW), lambda i: (0, i, 0)),
            pl.BlockSpec((_T, B2, _ROW), lambda i: (0, i, 0)),
            pl.BlockSpec((_T, B2, _H), lambda i: (0, i, 0)),
            pl.BlockSpec((1, _D_GAT), lambda i: (0, 0)),
            pl.BlockSpec((_D_GAT, 3 * _D_HID), lambda i: (0, 0)),
            pl.BlockSpec((_D_HID, 3 * _D_HID), lambda i: (0, 0)),
            pl.BlockSpec((1, 3 * _D_HID), lambda i: (0, 0)),
            pl.BlockSpec((1, 3 * _D_HID), lambda i: (0, 0)),
            pl.BlockSpec((_D_HID, 1), lambda i: (0, 0)),
            pl.BlockSpec((1, 1), lambda i: (0, 0)),
        ],
        out_specs=pl.BlockSpec((B2, 1), lambda i: (i, 0)),
        out_shape=jax.ShapeDtypeStruct((_N_PAD, 1), jnp.float32),
    )(out3d, srctab, adtab, b_gat, wihT, whhT, b_ih, b_hh, W_out, b_out)


def kernel(x, edge_index, W_emb, b_emb, W_gat, att_src, att_dst, b_gat,
           W_ih, W_hh, b_ih, b_hh, W_out, b_out):
    x2d = x.reshape(_N, _T * _F_IN)
    z16 = jnp.zeros((_C, 1), jnp.float32)
    As = jnp.concatenate([
        jnp.concatenate([att_src[0, 0][:, None], z16], axis=0),
        jnp.concatenate([z16, att_src[0, 1][:, None]], axis=0)], axis=1)
    Ad = jnp.concatenate([
        jnp.concatenate([att_dst[0, 0][:, None], z16], axis=0),
        jnp.concatenate([z16, att_dst[0, 1][:, None]], axis=0)], axis=1)

    src = edge_index[0]
    dst = edge_index[1]
    pad_i = jnp.arange(_E_PAD - _E, dtype=jnp.int32)
    src_p = jnp.concatenate([src, pad_i % _N])
    dst_p = jnp.concatenate([dst, _N + pad_i % (_N_PAD - _N)])
    src2d = src_p.reshape(_E_PAD // 128, 128)
    dst2d = dst_p.reshape(_E_PAD // 128, 128)
    zrows = jnp.zeros((_RPS, _ROW), jnp.float32)

    srctab, adtab = _build_tables(x2d, W_emb, b_emb[None, :], W_gat, As, Ad)
    srctab_flat = srctab.reshape(_T * _N, _ROW)
    adtab_flat = adtab.reshape(_T * _N, _H)

    out_flat = _sc_edge_pass(srctab_flat, adtab_flat, src2d, dst2d, zrows)
    out3d = out_flat.reshape(_T, _N_PAD, _ROW)

    pred = _final_stage(out3d, srctab, adtab, b_gat[None, :],
                        W_ih.T, W_hh.T, b_ih[None, :], b_hh[None, :],
                        W_out, b_out[None, :])
    return pred[:_N, 0]


# speedup vs baseline: 1.0530x; 1.0530x over previous
import functools

import jax
import jax.numpy as jnp
from jax import lax
from jax.experimental import pallas as pl
from jax.experimental.pallas import tpu as pltpu
from jax.experimental.pallas import tpu_sc as plsc

_N = 50000
_E = 800000
_T = 12
_F_IN = 2
_D_EMB = 32
_H = 2
_C = 16
_D_GAT = _H * _C
_D_HID = 16

_ROW = 34
_NSUB = 16
_NCORE = 2
_T_PER_CORE = _T // _NCORE
_N_PAD = 50176
_RPS = _N_PAD // (_NSUB * _NCORE)
_E_PAD = 819200
_KS = 256
_NSLOT = _E_PAD // _NSUB // _KS


def _tables_body(x_ref, wemb_ref, bemb_ref, wgat_ref, asrc_ref, adst_ref,
                 srctab_ref, adtab_ref):
    xb = x_ref[...]
    for t in range(_T):
        xt = xb[:, _F_IN * t:_F_IN * (t + 1)]
        emb = jnp.maximum(
            jnp.dot(xt, wemb_ref[...], preferred_element_type=jnp.float32)
            + bemb_ref[...], 0.0)
        xl = jnp.dot(emb, wgat_ref[...], preferred_element_type=jnp.float32)
        a_s = jnp.dot(xl, asrc_ref[...], preferred_element_type=jnp.float32)
        a_d = jnp.dot(xl, adst_ref[...], preferred_element_type=jnp.float32)
        srctab_ref[t, :, 0:_D_GAT] = xl
        srctab_ref[t, :, _D_GAT:_ROW] = a_s
        adtab_ref[t, :, :] = a_d


def _build_tables(x2d, W_emb, b_emb, W_gat, As, Ad):
    B1 = 512
    nb = pl.cdiv(_N, B1)
    return pl.pallas_call(
        _tables_body,
        grid=(nb,),
        in_specs=[
            pl.BlockSpec((B1, _T * _F_IN), lambda i: (i, 0)),
            pl.BlockSpec((_F_IN, _D_EMB), lambda i: (0, 0)),
            pl.BlockSpec((1, _D_EMB), lambda i: (0, 0)),
            pl.BlockSpec((_D_EMB, _D_GAT), lambda i: (0, 0)),
            pl.BlockSpec((_D_GAT, _H), lambda i: (0, 0)),
            pl.BlockSpec((_D_GAT, _H), lambda i: (0, 0)),
        ],
        out_specs=[
            pl.BlockSpec((_T, B1, _ROW), lambda i: (0, i, 0)),
            pl.BlockSpec((_T, B1, _H), lambda i: (0, i, 0)),
        ],
        out_shape=[
            jax.ShapeDtypeStruct((_T, _N, _ROW), jnp.float32),
            jax.ShapeDtypeStruct((_T, _N, _H), jnp.float32),
        ],
    )(x2d, W_emb, b_emb, W_gat, As, Ad)


def _sc_edge_pass(srctab_flat, adtab_flat, srcsh, dstsh, dstraw, zrows):
    mesh = plsc.VectorSubcoreMesh(core_axis_name="c", subcore_axis_name="s")

    @functools.partial(
        pl.kernel,
        out_type=jax.ShapeDtypeStruct((_T * _N_PAD, _ROW), jnp.float32),
        mesh=mesh,
        compiler_params=pltpu.CompilerParams(needs_layout_passes=False,
                                             use_tc_tiling_on_sc=False,
                                             disable_bounds_checks=True),
        scratch_types=[
            pltpu.VMEM((_KS,), jnp.int32),
            pltpu.VMEM((_KS,), jnp.int32),
            pltpu.VMEM((_KS,), jnp.int32),
            pltpu.VMEM((_KS, _ROW), jnp.float32),
            pltpu.VMEM((_KS, _H), jnp.float32),
            pltpu.VMEM_SHARED((_N_PAD, _ROW), jnp.float32),
            pltpu.SemaphoreType.DMA,
            pltpu.SemaphoreType.DMA,
            pltpu.SemaphoreType.DMA,
        ],
    )
    def k(srctab_hbm, adtab_hbm, ssh_hbm, dsh_hbm, draw_hbm, z_hbm,
          out_hbm, sidx, didx, draw, sbuf, adb, acc, semi, semg, sems):
        cid = lax.axis_index("c")
        sid = lax.axis_index("s")
        lanes = lax.iota(jnp.int32, 16)

        def per_t(tl, _):
            tg = cid * _T_PER_CORE + tl
            pltpu.sync_copy(z_hbm, acc.at[pl.ds(sid * _RPS, _RPS)])
            plsc.subcore_barrier()

            def slot_body(ci, _):
                @pl.when(ci > 0)
                def _():
                    pltpu.make_async_copy(sbuf, acc.at[draw], sems).wait()

                ioff = tg * _E_PAD + sid * (_E_PAD // _NSUB) + ci * _KS
                joff = sid * (_E_PAD // _NSUB) + ci * _KS
                c1 = pltpu.async_copy(ssh_hbm.at[pl.ds(ioff, _KS)], sidx, semi)
                c2 = pltpu.async_copy(dsh_hbm.at[pl.ds(ioff, _KS)], didx, semi)
                c3 = pltpu.async_copy(draw_hbm.at[pl.ds(joff, _KS)], draw, semi)
                c1.wait()
                c2.wait()
                c3.wait()

                g1 = pltpu.async_copy(srctab_hbm.at[sidx], sbuf, semg)
                g2 = pltpu.async_copy(adtab_hbm.at[didx], adb, semg)
                g1.wait()
                g2.wait()

                c32 = jnp.full((16,), _D_GAT, jnp.int32)
                c33 = jnp.full((16,), _D_GAT + 1, jnp.int32)
                h0 = jnp.full((16,), 0, jnp.int32)
                h1 = jnp.full((16,), 1, jnp.int32)

                def ex_body(i, _):
                    e16 = i * 16 + lanes
                    as0 = plsc.load_gather(sbuf, [e16, c32])
                    as1 = plsc.load_gather(sbuf, [e16, c33])
                    ad0 = plsc.load_gather(adb, [e16, h0])
                    ad1 = plsc.load_gather(adb, [e16, h1])
                    al0 = as0 + ad0
                    al1 = as1 + ad1
                    al0 = jnp.where(al0 >= 0.0, al0, 0.2 * al0)
                    al1 = jnp.where(al1 >= 0.0, al1, 0.2 * al1)
                    plsc.store_scatter(sbuf, [e16, c32], jnp.exp(al0))
                    plsc.store_scatter(sbuf, [e16, c33], jnp.exp(al1))
                    return 0

                lax.fori_loop(0, _KS // 16, ex_body, 0)

                zeros16 = jnp.zeros((16,), jnp.int32)

                def scale_body(i, _):
                    for u in range(4):
                        e = i * 4 + u
                        ev = zeros16 + e
                        s0 = plsc.load_gather(sbuf, [ev, c32])
                        s1 = plsc.load_gather(sbuf, [ev, c33])
                        sbuf[e, pl.ds(0, _C)] = sbuf[e, pl.ds(0, _C)] * s0
                        sbuf[e, pl.ds(_C, _C)] = sbuf[e, pl.ds(_C, _C)] * s1
                    return 0

                lax.fori_loop(0, _KS // 4, scale_body, 0)

                pltpu.async_copy(sbuf, acc.at[draw], sems, add=True)
                return 0

            lax.fori_loop(0, _NSLOT, slot_body, 0)
            pltpu.make_async_copy(sbuf, acc.at[draw], sems).wait()
            plsc.subcore_barrier()
            pltpu.sync_copy(
                acc.at[pl.ds(sid * _RPS, _RPS)],
                out_hbm.at[pl.ds(tg * _N_PAD + sid * _RPS, _RPS)])
            return 0

        lax.fori_loop(0, _T_PER_CORE, per_t, 0)

    return k(srctab_flat, adtab_flat, srcsh, dstsh, dstraw, zrows)


def _final_body(out_ref, srctab_ref, adtab_ref, bgat_ref, wihT_ref, whhT_ref,
                bih_ref, bhh_ref, wout_ref, bout_ref, pred_ref):
    B2 = pred_ref.shape[0]
    h = jnp.zeros((B2, _D_HID), jnp.float32)
    bgat = bgat_ref[...]
    for t in range(_T):
        row = out_ref[t]
        st = srctab_ref[t]
        num = row[:, 0:_D_GAT]
        den_e = row[:, _D_GAT:_ROW]
        xl = st[:, 0:_D_GAT]
        a_s = st[:, _D_GAT:_ROW]
        a_d = adtab_ref[t]
        alpha = a_s + a_d
        alpha = jnp.where(alpha >= 0.0, alpha, 0.2 * alpha)
        ex = jnp.exp(alpha)
        den = den_e + ex + 1e-16
        exb = jnp.concatenate(
            [jnp.broadcast_to(ex[:, 0:1], (B2, _C)),
             jnp.broadcast_to(ex[:, 1:2], (B2, _C))], axis=1)
        denb = jnp.concatenate(
            [jnp.broadcast_to(den[:, 0:1], (B2, _C)),
             jnp.broadcast_to(den[:, 1:2], (B2, _C))], axis=1)
        gat = (num + exb * xl) / denb + bgat
        gi = jnp.dot(gat, wihT_ref[...], preferred_element_type=jnp.float32) + bih_ref[...]
        gh = jnp.dot(h, whhT_ref[...], preferred_element_type=jnp.float32) + bhh_ref[...]
        r = jax.nn.sigmoid(gi[:, 0:_D_HID] + gh[:, 0:_D_HID])
        z = jax.nn.sigmoid(gi[:, _D_HID:2 * _D_HID] + gh[:, _D_HID:2 * _D_HID])
        ng = jnp.tanh(gi[:, 2 * _D_HID:3 * _D_HID] + r * gh[:, 2 * _D_HID:3 * _D_HID])
        h = (1.0 - z) * ng + z * h
    pred_ref[...] = jnp.dot(h, wout_ref[...], preferred_element_type=jnp.float32) + bout_ref[...]


def _final_stage(out3d, srctab, adtab, b_gat, wihT, whhT, b_ih, b_hh, W_out, b_out):
    B2 = 256
    nb = _N_PAD // B2
    return pl.pallas_call(
        _final_body,
        grid=(nb,),
        in_specs=[
            pl.BlockSpec((_T, B2, _ROW), lambda i: (0, i, 0)),
            pl.BlockSpec((_T, B2, _ROW), lambda i: (0, i, 0)),
            pl.BlockSpec((_T, B2, _H), lambda i: (0, i, 0)),
            pl.BlockSpec((1, _D_GAT), lambda i: (0, 0)),
            pl.BlockSpec((_D_GAT, 3 * _D_HID), lambda i: (0, 0)),
            pl.BlockSpec((_D_HID, 3 * _D_HID), lambda i: (0, 0)),
            pl.BlockSpec((1, 3 * _D_HID), lambda i: (0, 0)),
            pl.BlockSpec((1, 3 * _D_HID), lambda i: (0, 0)),
            pl.BlockSpec((_D_HID, 1), lambda i: (0, 0)),
            pl.BlockSpec((1, 1), lambda i: (0, 0)),
        ],
        out_specs=pl.BlockSpec((B2, 1), lambda i: (i, 0)),
        out_shape=jax.ShapeDtypeStruct((_N_PAD, 1), jnp.float32),
    )(out3d, srctab, adtab, b_gat, wihT, whhT, b_ih, b_hh, W_out, b_out)


def kernel(x, edge_index, W_emb, b_emb, W_gat, att_src, att_dst, b_gat,
           W_ih, W_hh, b_ih, b_hh, W_out, b_out):
    x2d = x.reshape(_N, _T * _F_IN)
    z16 = jnp.zeros((_C, 1), jnp.float32)
    As = jnp.concatenate([
        jnp.concatenate([att_src[0, 0][:, None], z16], axis=0),
        jnp.concatenate([z16, att_src[0, 1][:, None]], axis=0)], axis=1)
    Ad = jnp.concatenate([
        jnp.concatenate([att_dst[0, 0][:, None], z16], axis=0),
        jnp.concatenate([z16, att_dst[0, 1][:, None]], axis=0)], axis=1)

    src = edge_index[0]
    dst = edge_index[1]
    pad_i = jnp.arange(_E_PAD - _E, dtype=jnp.int32)
    src_p = jnp.concatenate([src, pad_i % _N])
    dst_p = jnp.concatenate([dst, _N + pad_i % (_N_PAD - _N)])
    tshift = (jnp.arange(_T, dtype=jnp.int32) * _N)[:, None]
    srcsh = (src_p[None, :] + tshift).reshape(_T * _E_PAD)
    dstsh = (dst_p[None, :] + tshift).reshape(_T * _E_PAD)
    dstraw = dst_p
    zrows = jnp.zeros((_RPS, _ROW), jnp.float32)

    srctab, adtab = _build_tables(x2d, W_emb, b_emb[None, :], W_gat, As, Ad)

    out_flat = _sc_edge_pass(srctab.reshape(_T * _N, _ROW),
                             adtab.reshape(_T * _N, _H),
                             srcsh, dstsh, dstraw, zrows)

    pred = _final_stage(out_flat.reshape(_T, _N_PAD, _ROW), srctab, adtab,
                        b_gat[None, :], W_ih.T, W_hh.T, b_ih[None, :],
                        b_hh[None, :], W_out, b_out[None, :])
    return pred[:_N, 0]


# speedup vs baseline: 1.2903x; 1.2253x over previous
import functools

import jax
import jax.numpy as jnp
from jax import lax
from jax.experimental import pallas as pl
from jax.experimental.pallas import tpu as pltpu
from jax.experimental.pallas import tpu_sc as plsc

_N = 50000
_E = 800000
_T = 12
_F_IN = 2
_D_EMB = 32
_H = 2
_C = 16
_D_GAT = _H * _C
_D_HID = 16

_ROW = 34
_NSUB = 16
_NCORE = 2
_T_PER_CORE = _T // _NCORE
_N_PAD = 50176
_RPS = _N_PAD // (_NSUB * _NCORE)
_E_PAD = 819200
_KS = 256
_NSLOT = _E_PAD // _NSUB // _KS
_BLK = 8


def _tables_body(x_ref, wemb_ref, bemb_ref, wgat_ref, asrc_ref, adst_ref,
                 srctab_ref, adtab_ref):
    xb = x_ref[...]
    for t in range(_T):
        xt = xb[:, _F_IN * t:_F_IN * (t + 1)]
        emb = jnp.maximum(
            jnp.dot(xt, wemb_ref[...], preferred_element_type=jnp.float32)
            + bemb_ref[...], 0.0)
        xl = jnp.dot(emb, wgat_ref[...], preferred_element_type=jnp.float32)
        a_s = jnp.dot(xl, asrc_ref[...], preferred_element_type=jnp.float32)
        a_d = jnp.dot(xl, adst_ref[...], preferred_element_type=jnp.float32)
        srctab_ref[t, :, 0:_D_GAT] = xl
        srctab_ref[t, :, _D_GAT:_ROW] = a_s
        adtab_ref[t, :, :] = a_d


def _build_tables(x2d, W_emb, b_emb, W_gat, As, Ad):
    B1 = 512
    nb = pl.cdiv(_N, B1)
    return pl.pallas_call(
        _tables_body,
        grid=(nb,),
        in_specs=[
            pl.BlockSpec((B1, _T * _F_IN), lambda i: (i, 0)),
            pl.BlockSpec((_F_IN, _D_EMB), lambda i: (0, 0)),
            pl.BlockSpec((1, _D_EMB), lambda i: (0, 0)),
            pl.BlockSpec((_D_EMB, _D_GAT), lambda i: (0, 0)),
            pl.BlockSpec((_D_GAT, _H), lambda i: (0, 0)),
            pl.BlockSpec((_D_GAT, _H), lambda i: (0, 0)),
        ],
        out_specs=[
            pl.BlockSpec((_T, B1, _ROW), lambda i: (0, i, 0)),
            pl.BlockSpec((_T, B1, _H), lambda i: (0, i, 0)),
        ],
        out_shape=[
            jax.ShapeDtypeStruct((_T, _N, _ROW), jnp.float32),
            jax.ShapeDtypeStruct((_T, _N, _H), jnp.float32),
        ],
    )(x2d, W_emb, b_emb, W_gat, As, Ad)


def _sc_edge_pass(srctab_flat, adtab_flat, srcsh, dstsh, dstraw, zrows):
    mesh = plsc.VectorSubcoreMesh(core_axis_name="c", subcore_axis_name="s")

    @functools.partial(
        pl.kernel,
        out_type=jax.ShapeDtypeStruct((_T * _N_PAD, _ROW), jnp.float32),
        mesh=mesh,
        compiler_params=pltpu.CompilerParams(needs_layout_passes=False,
                                             use_tc_tiling_on_sc=False,
                                             disable_bounds_checks=True),
        scratch_types=[
            pltpu.VMEM((_BLK, _KS), jnp.int32),
            pltpu.VMEM((_BLK, _KS), jnp.int32),
            pltpu.VMEM((_BLK, _KS), jnp.int32),
            pltpu.VMEM((_KS, _ROW), jnp.float32),
            pltpu.VMEM((_KS, _H), jnp.float32),
            pltpu.VMEM_SHARED((_N_PAD, _ROW), jnp.float32),
            pltpu.SemaphoreType.DMA,
            pltpu.SemaphoreType.DMA,
            pltpu.SemaphoreType.DMA,
        ],
    )
    def k(srctab_hbm, adtab_hbm, ssh_hbm, dsh_hbm, draw_hbm, z_hbm,
          out_hbm, sblk, dblk, rblk, sbuf, adb, acc, semi, semg, sems):
        cid = lax.axis_index("c")
        sid = lax.axis_index("s")
        lanes = lax.iota(jnp.int32, 16)

        def per_t(tl, _):
            tg = cid * _T_PER_CORE + tl
            pltpu.sync_copy(z_hbm, acc.at[pl.ds(sid * _RPS, _RPS)])
            plsc.subcore_barrier()

            def slot_body(ci, _):
                r = lax.rem(ci, _BLK)

                @pl.when(ci > 0)
                def _():
                    pltpu.make_async_copy(sbuf, acc.at[rblk.at[0]], sems).wait()

                @pl.when(r == 0)
                def _():
                    irow = tg * (_E_PAD // _KS) + sid * _NSLOT + ci
                    jrow = sid * _NSLOT + ci
                    c1 = pltpu.async_copy(ssh_hbm.at[pl.ds(irow, _BLK)], sblk, semi)
                    c2 = pltpu.async_copy(dsh_hbm.at[pl.ds(irow, _BLK)], dblk, semi)
                    c3 = pltpu.async_copy(draw_hbm.at[pl.ds(jrow, _BLK)], rblk, semi)
                    c1.wait()
                    c2.wait()
                    c3.wait()

                g1 = pltpu.async_copy(srctab_hbm.at[sblk.at[r]], sbuf, semg)
                g2 = pltpu.async_copy(adtab_hbm.at[dblk.at[r]], adb, semg)
                g1.wait()
                g2.wait()

                c32 = jnp.full((16,), _D_GAT, jnp.int32)
                c33 = jnp.full((16,), _D_GAT + 1, jnp.int32)
                h0 = jnp.full((16,), 0, jnp.int32)
                h1 = jnp.full((16,), 1, jnp.int32)

                def ex_body(i, _):
                    e16 = i * 16 + lanes
                    as0 = plsc.load_gather(sbuf, [e16, c32])
                    as1 = plsc.load_gather(sbuf, [e16, c33])
                    ad0 = plsc.load_gather(adb, [e16, h0])
                    ad1 = plsc.load_gather(adb, [e16, h1])
                    al0 = as0 + ad0
                    al1 = as1 + ad1
                    al0 = jnp.where(al0 >= 0.0, al0, 0.2 * al0)
                    al1 = jnp.where(al1 >= 0.0, al1, 0.2 * al1)
                    plsc.store_scatter(sbuf, [e16, c32], jnp.exp(al0))
                    plsc.store_scatter(sbuf, [e16, c33], jnp.exp(al1))
                    return 0

                lax.fori_loop(0, _KS // 16, ex_body, 0)

                zeros16 = jnp.zeros((16,), jnp.int32)

                def scale_body(i, _):
                    for u in range(4):
                        e = i * 4 + u
                        ev = zeros16 + e
                        s0 = plsc.load_gather(sbuf, [ev, c32])
                        s1 = plsc.load_gather(sbuf, [ev, c33])
                        sbuf[e, pl.ds(0, _C)] = sbuf[e, pl.ds(0, _C)] * s0
                        sbuf[e, pl.ds(_C, _C)] = sbuf[e, pl.ds(_C, _C)] * s1
                    return 0

                lax.fori_loop(0, _KS // 4, scale_body, 0)

                pltpu.async_copy(sbuf, acc.at[rblk.at[r]], sems, add=True)
                return 0

            lax.fori_loop(0, _NSLOT, slot_body, 0)
            pltpu.make_async_copy(sbuf, acc.at[rblk.at[0]], sems).wait()
            plsc.subcore_barrier()
            pltpu.sync_copy(
                acc.at[pl.ds(sid * _RPS, _RPS)],
                out_hbm.at[pl.ds(tg * _N_PAD + sid * _RPS, _RPS)])
            return 0

        lax.fori_loop(0, _T_PER_CORE, per_t, 0)

    return k(srctab_flat, adtab_flat, srcsh, dstsh, dstraw, zrows)


def _final_body(out_ref, srctab_ref, adtab_ref, bgat_ref, wihT_ref, whhT_ref,
                bih_ref, bhh_ref, wout_ref, bout_ref, pred_ref):
    B2 = pred_ref.shape[0]
    h = jnp.zeros((B2, _D_HID), jnp.float32)
    bgat = bgat_ref[...]
    for t in range(_T):
        row = out_ref[t]
        st = srctab_ref[t]
        num = row[:, 0:_D_GAT]
        den_e = row[:, _D_GAT:_ROW]
        xl = st[:, 0:_D_GAT]
        a_s = st[:, _D_GAT:_ROW]
        a_d = adtab_ref[t]
        alpha = a_s + a_d
        alpha = jnp.where(alpha >= 0.0, alpha, 0.2 * alpha)
        ex = jnp.exp(alpha)
        den = den_e + ex + 1e-16
        exb = jnp.concatenate(
            [jnp.broadcast_to(ex[:, 0:1], (B2, _C)),
             jnp.broadcast_to(ex[:, 1:2], (B2, _C))], axis=1)
        denb = jnp.concatenate(
            [jnp.broadcast_to(den[:, 0:1], (B2, _C)),
             jnp.broadcast_to(den[:, 1:2], (B2, _C))], axis=1)
        gat = (num + exb * xl) / denb + bgat
        gi = jnp.dot(gat, wihT_ref[...], preferred_element_type=jnp.float32) + bih_ref[...]
        gh = jnp.dot(h, whhT_ref[...], preferred_element_type=jnp.float32) + bhh_ref[...]
        r = jax.nn.sigmoid(gi[:, 0:_D_HID] + gh[:, 0:_D_HID])
        z = jax.nn.sigmoid(gi[:, _D_HID:2 * _D_HID] + gh[:, _D_HID:2 * _D_HID])
        ng = jnp.tanh(gi[:, 2 * _D_HID:3 * _D_HID] + r * gh[:, 2 * _D_HID:3 * _D_HID])
        h = (1.0 - z) * ng + z * h
    pred_ref[...] = jnp.dot(h, wout_ref[...], preferred_element_type=jnp.float32) + bout_ref[...]


def _final_stage(out3d, srctab, adtab, b_gat, wihT, whhT, b_ih, b_hh, W_out, b_out):
    B2 = 256
    nb = _N_PAD // B2
    return pl.pallas_call(
        _final_body,
        grid=(nb,),
        in_specs=[
            pl.BlockSpec((_T, B2, _ROW), lambda i: (0, i, 0)),
            pl.BlockSpec((_T, B2, _ROW), lambda i: (0, i, 0)),
            pl.BlockSpec((_T, B2, _H), lambda i: (0, i, 0)),
            pl.BlockSpec((1, _D_GAT), lambda i: (0, 0)),
            pl.BlockSpec((_D_GAT, 3 * _D_HID), lambda i: (0, 0)),
            pl.BlockSpec((_D_HID, 3 * _D_HID), lambda i: (0, 0)),
            pl.BlockSpec((1, 3 * _D_HID), lambda i: (0, 0)),
            pl.BlockSpec((1, 3 * _D_HID), lambda i: (0, 0)),
            pl.BlockSpec((_D_HID, 1), lambda i: (0, 0)),
            pl.BlockSpec((1, 1), lambda i: (0, 0)),
        ],
        out_specs=pl.BlockSpec((B2, 1), lambda i: (i, 0)),
        out_shape=jax.ShapeDtypeStruct((_N_PAD, 1), jnp.float32),
    )(out3d, srctab, adtab, b_gat, wihT, whhT, b_ih, b_hh, W_out, b_out)


def kernel(x, edge_index, W_emb, b_emb, W_gat, att_src, att_dst, b_gat,
           W_ih, W_hh, b_ih, b_hh, W_out, b_out):
    x2d = x.reshape(_N, _T * _F_IN)
    z16 = jnp.zeros((_C, 1), jnp.float32)
    As = jnp.concatenate([
        jnp.concatenate([att_src[0, 0][:, None], z16], axis=0),
        jnp.concatenate([z16, att_src[0, 1][:, None]], axis=0)], axis=1)
    Ad = jnp.concatenate([
        jnp.concatenate([att_dst[0, 0][:, None], z16], axis=0),
        jnp.concatenate([z16, att_dst[0, 1][:, None]], axis=0)], axis=1)

    src = edge_index[0]
    dst = edge_index[1]
    pad_i = jnp.arange(_E_PAD - _E, dtype=jnp.int32)
    src_p = jnp.concatenate([src, pad_i % _N])
    dst_p = jnp.concatenate([dst, _N + pad_i % (_N_PAD - _N)])
    tshift = (jnp.arange(_T, dtype=jnp.int32) * _N)[:, None]
    srcsh = (src_p[None, :] + tshift).reshape(_T * _E_PAD // _KS, _KS)
    dstsh = (dst_p[None, :] + tshift).reshape(_T * _E_PAD // _KS, _KS)
    dstraw = dst_p.reshape(_E_PAD // _KS, _KS)
    zrows = jnp.zeros((_RPS, _ROW), jnp.float32)

    srctab, adtab = _build_tables(x2d, W_emb, b_emb[None, :], W_gat, As, Ad)

    out_flat = _sc_edge_pass(srctab.reshape(_T * _N, _ROW),
                             adtab.reshape(_T * _N, _H),
                             srcsh, dstsh, dstraw, zrows)

    pred = _final_stage(out_flat.reshape(_T, _N_PAD, _ROW), srctab, adtab,
                        b_gat[None, :], W_ih.T, W_hh.T, b_ih[None, :],
                        b_hh[None, :], W_out, b_out[None, :])
    return pred[:_N, 0]


# speedup vs baseline: 1.3975x; 1.0831x over previous
import functools

import jax
import jax.numpy as jnp
from jax import lax
from jax.experimental import pallas as pl
from jax.experimental.pallas import tpu as pltpu
from jax.experimental.pallas import tpu_sc as plsc

_N = 50000
_E = 800000
_T = 12
_F_IN = 2
_D_EMB = 32
_H = 2
_C = 16
_D_GAT = _H * _C
_D_HID = 16

_ROW = 34
_NSUB = 16
_NCORE = 2
_T_PER_CORE = _T // _NCORE
_N_PAD = 50176
_RPS = _N_PAD // (_NSUB * _NCORE)
_E_PAD = 819200
_KS = 256
_NSLOT = _E_PAD // _NSUB // _KS
_BLK = 8


def _tables_body(x_ref, wemb_ref, bemb_ref, wgat_ref, asrc_ref, adst_ref,
                 srctab_ref, adtab_ref):
    xb = x_ref[...]
    for t in range(_T):
        xt = xb[:, _F_IN * t:_F_IN * (t + 1)]
        emb = jnp.maximum(
            jnp.dot(xt, wemb_ref[...], preferred_element_type=jnp.float32)
            + bemb_ref[...], 0.0)
        xl = jnp.dot(emb, wgat_ref[...], preferred_element_type=jnp.float32)
        a_s = jnp.dot(xl, asrc_ref[...], preferred_element_type=jnp.float32)
        a_d = jnp.dot(xl, adst_ref[...], preferred_element_type=jnp.float32)
        srctab_ref[t, :, 0:_D_GAT] = xl
        srctab_ref[t, :, _D_GAT:_ROW] = a_s
        adtab_ref[t, :, :] = a_d


def _build_tables(x2d, W_emb, b_emb, W_gat, As, Ad):
    B1 = 512
    nb = pl.cdiv(_N, B1)
    return pl.pallas_call(
        _tables_body,
        grid=(nb,),
        in_specs=[
            pl.BlockSpec((B1, _T * _F_IN), lambda i: (i, 0)),
            pl.BlockSpec((_F_IN, _D_EMB), lambda i: (0, 0)),
            pl.BlockSpec((1, _D_EMB), lambda i: (0, 0)),
            pl.BlockSpec((_D_EMB, _D_GAT), lambda i: (0, 0)),
            pl.BlockSpec((_D_GAT, _H), lambda i: (0, 0)),
            pl.BlockSpec((_D_GAT, _H), lambda i: (0, 0)),
        ],
        out_specs=[
            pl.BlockSpec((_T, B1, _ROW), lambda i: (0, i, 0)),
            pl.BlockSpec((_T, B1, _H), lambda i: (0, i, 0)),
        ],
        out_shape=[
            jax.ShapeDtypeStruct((_T, _N, _ROW), jnp.float32),
            jax.ShapeDtypeStruct((_T, _N, _H), jnp.float32),
        ],
    )(x2d, W_emb, b_emb, W_gat, As, Ad)


def _sc_edge_pass(srctab_flat, adtab_flat, srcsh, dstsh, dstraw, zrows):
    mesh = plsc.VectorSubcoreMesh(core_axis_name="c", subcore_axis_name="s")

    @functools.partial(
        pl.kernel,
        out_type=jax.ShapeDtypeStruct((_T * _N_PAD, _ROW), jnp.float32),
        mesh=mesh,
        compiler_params=pltpu.CompilerParams(needs_layout_passes=False,
                                             use_tc_tiling_on_sc=False,
                                             disable_bounds_checks=True),
        scratch_types=[
            pltpu.VMEM((2 * _BLK, 128), jnp.int32),
            pltpu.VMEM((2 * _BLK, 128), jnp.int32),
            pltpu.VMEM((_BLK, _KS), jnp.int32),
            pltpu.VMEM((_KS, _ROW), jnp.float32),
            pltpu.VMEM((_KS, _H), jnp.float32),
            pltpu.VMEM_SHARED((_N_PAD, _ROW), jnp.float32),
            pltpu.SemaphoreType.DMA,
            pltpu.SemaphoreType.DMA,
            pltpu.SemaphoreType.DMA,
            pltpu.SemaphoreType.DMA,
        ],
    )
    def k(srctab_hbm, adtab_hbm, ssh_hbm, dsh_hbm, draw_hbm, z_hbm,
          out_hbm, sblk, dblk, rblk, sbuf, adb, acc, semi, semg, semh, sems):
        cid = lax.axis_index("c")
        sid = lax.axis_index("s")
        lanes = lax.iota(jnp.int32, 16)

        def per_t(tl, _):
            tg = cid * _T_PER_CORE + tl
            pltpu.sync_copy(z_hbm, acc.at[pl.ds(sid * _RPS, _RPS)])
            plsc.subcore_barrier()

            def slot_body(ci, _):
                r = lax.rem(ci, _BLK)

                @pl.when(ci > 0)
                def _():
                    pltpu.make_async_copy(sbuf, acc.at[rblk.at[0]], sems).wait()

                @pl.when(r == 0)
                def _():
                    irow = 2 * (tg * (_E_PAD // _KS) + sid * _NSLOT + ci)
                    jrow = sid * _NSLOT + ci
                    c1 = pltpu.async_copy(ssh_hbm.at[pl.ds(irow, 2 * _BLK)], sblk, semi)
                    c2 = pltpu.async_copy(dsh_hbm.at[pl.ds(irow, 2 * _BLK)], dblk, semi)
                    c3 = pltpu.async_copy(draw_hbm.at[pl.ds(jrow, _BLK)], rblk, semi)
                    c1.wait()
                    c2.wait()
                    c3.wait()

                g1a = pltpu.async_copy(srctab_hbm.at[sblk.at[2 * r]],
                                       sbuf.at[pl.ds(0, 128)], semg)
                g2a = pltpu.async_copy(adtab_hbm.at[dblk.at[2 * r]],
                                       adb.at[pl.ds(0, 128)], semg)
                g1b = pltpu.async_copy(srctab_hbm.at[sblk.at[2 * r + 1]],
                                       sbuf.at[pl.ds(128, 128)], semh)
                g2b = pltpu.async_copy(adtab_hbm.at[dblk.at[2 * r + 1]],
                                       adb.at[pl.ds(128, 128)], semh)

                c32 = jnp.full((16,), _D_GAT, jnp.int32)
                c33 = jnp.full((16,), _D_GAT + 1, jnp.int32)
                h0 = jnp.full((16,), 0, jnp.int32)
                h1 = jnp.full((16,), 1, jnp.int32)
                zeros16 = jnp.zeros((16,), jnp.int32)

                def ex_body(i, _):
                    e16 = i * 16 + lanes
                    as0 = plsc.load_gather(sbuf, [e16, c32])
                    as1 = plsc.load_gather(sbuf, [e16, c33])
                    ad0 = plsc.load_gather(adb, [e16, h0])
                    ad1 = plsc.load_gather(adb, [e16, h1])
                    al0 = as0 + ad0
                    al1 = as1 + ad1
                    al0 = jnp.where(al0 >= 0.0, al0, 0.2 * al0)
                    al1 = jnp.where(al1 >= 0.0, al1, 0.2 * al1)
                    plsc.store_scatter(sbuf, [e16, c32], jnp.exp(al0))
                    plsc.store_scatter(sbuf, [e16, c33], jnp.exp(al1))
                    return 0

                def scale_body(i, _):
                    for u in range(4):
                        e = i * 4 + u
                        ev = zeros16 + e
                        s0 = plsc.load_gather(sbuf, [ev, c32])
                        s1 = plsc.load_gather(sbuf, [ev, c33])
                        sbuf[e, pl.ds(0, _C)] = sbuf[e, pl.ds(0, _C)] * s0
                        sbuf[e, pl.ds(_C, _C)] = sbuf[e, pl.ds(_C, _C)] * s1
                    return 0

                g1a.wait()
                g2a.wait()
                lax.fori_loop(0, 128 // 16, ex_body, 0)
                lax.fori_loop(0, 128 // 4, scale_body, 0)
                g1b.wait()
                g2b.wait()
                lax.fori_loop(128 // 16, _KS // 16, ex_body, 0)
                lax.fori_loop(128 // 4, _KS // 4, scale_body, 0)

                pltpu.async_copy(sbuf, acc.at[rblk.at[r]], sems, add=True)
                return 0

            lax.fori_loop(0, _NSLOT, slot_body, 0)
            pltpu.make_async_copy(sbuf, acc.at[rblk.at[0]], sems).wait()
            plsc.subcore_barrier()
            pltpu.sync_copy(
                acc.at[pl.ds(sid * _RPS, _RPS)],
                out_hbm.at[pl.ds(tg * _N_PAD + sid * _RPS, _RPS)])
            return 0

        lax.fori_loop(0, _T_PER_CORE, per_t, 0)

    return k(srctab_flat, adtab_flat, srcsh, dstsh, dstraw, zrows)


def _final_body(out_ref, srctab_ref, adtab_ref, bgat_ref, wihT_ref, whhT_ref,
                bih_ref, bhh_ref, wout_ref, bout_ref, pred_ref):
    B2 = pred_ref.shape[0]
    h = jnp.zeros((B2, _D_HID), jnp.float32)
    bgat = bgat_ref[...]
    for t in range(_T):
        row = out_ref[t]
        st = srctab_ref[t]
        num = row[:, 0:_D_GAT]
        den_e = row[:, _D_GAT:_ROW]
        xl = st[:, 0:_D_GAT]
        a_s = st[:, _D_GAT:_ROW]
        a_d = adtab_ref[t]
        alpha = a_s + a_d
        alpha = jnp.where(alpha >= 0.0, alpha, 0.2 * alpha)
        ex = jnp.exp(alpha)
        den = den_e + ex + 1e-16
        exb = jnp.concatenate(
            [jnp.broadcast_to(ex[:, 0:1], (B2, _C)),
             jnp.broadcast_to(ex[:, 1:2], (B2, _C))], axis=1)
        denb = jnp.concatenate(
            [jnp.broadcast_to(den[:, 0:1], (B2, _C)),
             jnp.broadcast_to(den[:, 1:2], (B2, _C))], axis=1)
        gat = (num + exb * xl) / denb + bgat
        gi = jnp.dot(gat, wihT_ref[...], preferred_element_type=jnp.float32) + bih_ref[...]
        gh = jnp.dot(h, whhT_ref[...], preferred_element_type=jnp.float32) + bhh_ref[...]
        r = jax.nn.sigmoid(gi[:, 0:_D_HID] + gh[:, 0:_D_HID])
        z = jax.nn.sigmoid(gi[:, _D_HID:2 * _D_HID] + gh[:, _D_HID:2 * _D_HID])
        ng = jnp.tanh(gi[:, 2 * _D_HID:3 * _D_HID] + r * gh[:, 2 * _D_HID:3 * _D_HID])
        h = (1.0 - z) * ng + z * h
    pred_ref[...] = jnp.dot(h, wout_ref[...], preferred_element_type=jnp.float32) + bout_ref[...]


def _final_stage(out3d, srctab, adtab, b_gat, wihT, whhT, b_ih, b_hh, W_out, b_out):
    B2 = 256
    nb = _N_PAD // B2
    return pl.pallas_call(
        _final_body,
        grid=(nb,),
        in_specs=[
            pl.BlockSpec((_T, B2, _ROW), lambda i: (0, i, 0)),
            pl.BlockSpec((_T, B2, _ROW), lambda i: (0, i, 0)),
            pl.BlockSpec((_T, B2, _H), lambda i: (0, i, 0)),
            pl.BlockSpec((1, _D_GAT), lambda i: (0, 0)),
            pl.BlockSpec((_D_GAT, 3 * _D_HID), lambda i: (0, 0)),
            pl.BlockSpec((_D_HID, 3 * _D_HID), lambda i: (0, 0)),
            pl.BlockSpec((1, 3 * _D_HID), lambda i: (0, 0)),
            pl.BlockSpec((1, 3 * _D_HID), lambda i: (0, 0)),
            pl.BlockSpec((_D_HID, 1), lambda i: (0, 0)),
            pl.BlockSpec((1, 1), lambda i: (0, 0)),
        ],
        out_specs=pl.BlockSpec((B2, 1), lambda i: (i, 0)),
        out_shape=jax.ShapeDtypeStruct((_N_PAD, 1), jnp.float32),
    )(out3d, srctab, adtab, b_gat, wihT, whhT, b_ih, b_hh, W_out, b_out)


def kernel(x, edge_index, W_emb, b_emb, W_gat, att_src, att_dst, b_gat,
           W_ih, W_hh, b_ih, b_hh, W_out, b_out):
    x2d = x.reshape(_N, _T * _F_IN)
    z16 = jnp.zeros((_C, 1), jnp.float32)
    As = jnp.concatenate([
        jnp.concatenate([att_src[0, 0][:, None], z16], axis=0),
        jnp.concatenate([z16, att_src[0, 1][:, None]], axis=0)], axis=1)
    Ad = jnp.concatenate([
        jnp.concatenate([att_dst[0, 0][:, None], z16], axis=0),
        jnp.concatenate([z16, att_dst[0, 1][:, None]], axis=0)], axis=1)

    src = edge_index[0]
    dst = edge_index[1]
    pad_i = jnp.arange(_E_PAD - _E, dtype=jnp.int32)
    src_p = jnp.concatenate([src, pad_i % _N])
    dst_p = jnp.concatenate([dst, _N + pad_i % (_N_PAD - _N)])
    tshift = (jnp.arange(_T, dtype=jnp.int32) * _N)[:, None]
    srcsh = (src_p[None, :] + tshift).reshape(_T * _E_PAD // 128, 128)
    dstsh = (dst_p[None, :] + tshift).reshape(_T * _E_PAD // 128, 128)
    dstraw = dst_p.reshape(_E_PAD // _KS, _KS)
    zrows = jnp.zeros((_RPS, _ROW), jnp.float32)

    srctab, adtab = _build_tables(x2d, W_emb, b_emb[None, :], W_gat, As, Ad)

    out_flat = _sc_edge_pass(srctab.reshape(_T * _N, _ROW),
                             adtab.reshape(_T * _N, _H),
                             srcsh, dstsh, dstraw, zrows)

    pred = _final_stage(out_flat.reshape(_T, _N_PAD, _ROW), srctab, adtab,
                        b_gat[None, :], W_ih.T, W_hh.T, b_ih[None, :],
                        b_hh[None, :], W_out, b_out[None, :])
    return pred[:_N, 0]
